# baseline (speedup 1.0000x reference)
import jax
import jax.numpy as jnp
from jax import lax
from jax.experimental import pallas as pl
from jax.experimental.pallas import tpu as pltpu
from jax.experimental.pallas import tpu_sc as plsc

N = 10000
E = 320000
F = 128
H = 128
O = 64

NC = 2
NS = 16
NW = NC * NS
K = 128
EW = E // NW
CH = 80
EWP = CH * K
E_PAD = NW * EWP
N_PAD = 10112
RT = N_PAD // NS

import functools


@functools.lru_cache(maxsize=None)
def _mesh():
    return plsc.VectorSubcoreMesh(
        core_axis_name="c", subcore_axis_name="s",
        num_cores=NC, num_subcores=NS)


def _deg_body(dst_hbm, zeros16_hbm, ones_hbm, out_hbm, idx_v, ones_v, acc):
    cid = lax.axis_index("c")
    sid = lax.axis_index("s")
    wid = sid * NC + cid
    pltpu.sync_copy(dst_hbm.at[wid], idx_v)
    pltpu.sync_copy(ones_hbm, ones_v)
    pltpu.sync_copy(zeros16_hbm.at[pl.ds(sid * RT, RT)],
                    acc.at[pl.ds(sid * RT, RT)])
    plsc.subcore_barrier()

    def body(j, carry):
        pltpu.sync_copy(ones_v, acc.at[idx_v.at[j]], add=True)
        return carry

    lax.fori_loop(0, CH, body, 0)
    plsc.subcore_barrier()
    pltpu.sync_copy(acc.at[pl.ds(sid * RT, RT)],
                    out_hbm.at[cid, pl.ds(sid * RT, RT)])


@functools.lru_cache(maxsize=None)
def _deg_call():
    return pl.kernel(
        _deg_body,
        out_type=jax.ShapeDtypeStruct((NC, N_PAD, 16), jnp.float32),
        mesh=_mesh(),
        compiler_params=pltpu.CompilerParams(use_tc_tiling_on_sc=False),
        scratch_types=[
            pltpu.VMEM((CH, K), jnp.int32),
            pltpu.VMEM((K, 16), jnp.float32),
            pltpu.VMEM_SHARED((N_PAD, 16), jnp.float32),
        ],
    )


def _make_seg_call(d, k, ch):
    def body(y_hbm, src_hbm, dst_hbm, zeros_hbm, out_hbm,
             sidx, didx, rows0, rows1, sem0, sem1, acc):
        cid = lax.axis_index("c")
        sid = lax.axis_index("s")
        wid = sid * NC + cid
        pltpu.sync_copy(src_hbm.at[wid], sidx)
        pltpu.sync_copy(dst_hbm.at[wid], didx)
        pltpu.sync_copy(zeros_hbm.at[pl.ds(sid * RT, RT)],
                        acc.at[pl.ds(sid * RT, RT)])
        plsc.subcore_barrier()

        pltpu.async_copy(y_hbm.at[sidx.at[0]], rows0, sem0)
        pltpu.async_copy(y_hbm.at[sidx.at[1]], rows1, sem1)

        def step(jj, carry):
            for b, (rows, sem) in enumerate(((rows0, sem0), (rows1, sem1))):
                c = jj * 2 + b
                pltpu.make_async_copy(y_hbm.at[sidx.at[c]], rows, sem).wait()
                pltpu.sync_copy(rows, acc.at[didx.at[c]], add=True)

                @pl.when(c + 2 < ch)
                def _():
                    pltpu.async_copy(y_hbm.at[sidx.at[c + 2]], rows, sem)
            return carry

        lax.fori_loop(0, ch // 2, step, 0)
        plsc.subcore_barrier()
        pltpu.sync_copy(acc.at[pl.ds(sid * RT, RT)],
                        out_hbm.at[cid, pl.ds(sid * RT, RT)])

    return pl.kernel(
        body,
        out_type=jax.ShapeDtypeStruct((NC, N_PAD, d), jnp.float32),
        mesh=_mesh(),
        compiler_params=pltpu.CompilerParams(use_tc_tiling_on_sc=False),
        scratch_types=[
            pltpu.VMEM((ch, k), jnp.int32),
            pltpu.VMEM((ch, k), jnp.int32),
            pltpu.VMEM((k, d), jnp.float32),
            pltpu.VMEM((k, d), jnp.float32),
            pltpu.SemaphoreType.DMA,
            pltpu.SemaphoreType.DMA,
            pltpu.VMEM_SHARED((N_PAD, d), jnp.float32),
        ],
    )


_make_seg_call = functools.lru_cache(maxsize=None)(_make_seg_call)


def _tc1_body(x_ref, w_ref, degp_ref, y_ref, dinv_ref):
    deg = degp_ref[0][:, 0:1] + degp_ref[1][:, 0:1] + 1.0
    dinv = lax.rsqrt(deg)
    y_ref[...] = dinv * jnp.dot(x_ref[...], w_ref[...],
                                preferred_element_type=jnp.float32)
    dinv_ref[...] = dinv


_tc1_call = pl.pallas_call(
    _tc1_body,
    out_shape=(
        jax.ShapeDtypeStruct((N_PAD, H), jnp.float32),
        jax.ShapeDtypeStruct((N_PAD, 1), jnp.float32),
    ),
)


def _tc2_body(s_ref, y1_ref, dinv_ref, b1_ref, scale_ref, w2_ref, y2_ref):
    dinv = dinv_ref[...]
    h = dinv * (s_ref[0] + s_ref[1] + y1_ref[...]) + b1_ref[...]
    h = jnp.where(h >= 0.0, h, 0.01 * h)
    h = h * scale_ref[...]
    y2_ref[...] = dinv * jnp.dot(h, w2_ref[...],
                                 preferred_element_type=jnp.float32)


_tc2_call = pl.pallas_call(
    _tc2_body,
    out_shape=jax.ShapeDtypeStruct((N_PAD, O), jnp.float32),
)


def _tc3_body(s_ref, y2_ref, dinv_ref, b2_ref, out_ref):
    t = dinv_ref[...] * (s_ref[0] + s_ref[1] + y2_ref[...]) + b2_ref[...]
    out_ref[...] = jnp.where(t >= 0.0, t, 0.01 * t)


_tc3_call = pl.pallas_call(
    _tc3_body,
    out_shape=jax.ShapeDtypeStruct((N_PAD, O), jnp.float32),
)


def kernel(x, edge_idx, W1, b1, W2, b2):
    src = edge_idx[0]
    dst = edge_idx[1]
    padi = jnp.full((E_PAD - E,), N, jnp.int32)
    srcf = jnp.concatenate([src, padi])
    dstf = jnp.concatenate([dst, padi])
    srcp = srcf.reshape(NW, CH, K)
    dstp = dstf.reshape(NW, CH, K)
    srcp1 = srcf.reshape(NW, 2 * CH, K // 2)
    dstp1 = dstf.reshape(NW, 2 * CH, K // 2)
    xp = jnp.pad(x, ((0, N_PAD - N), (0, 0)))

    zeros16 = jnp.zeros((N_PAD, 16), jnp.float32)
    ones16 = jnp.ones((K, 16), jnp.float32)
    zerosH = jnp.zeros((N_PAD, H), jnp.float32)
    zerosO = jnp.zeros((N_PAD, O), jnp.float32)

    mask = jax.random.bernoulli(jax.random.key(42), 0.5, (N, H))
    scale = jnp.pad(jnp.where(mask, 2.0, 0.0).astype(jnp.float32),
                    ((0, N_PAD - N), (0, 0)))

    degp = _deg_call()(dstp, zeros16, ones16)
    y1, dinv = _tc1_call(xp, W1, degp)
    s1 = _make_seg_call(H, K // 2, 2 * CH)(y1, srcp1, dstp1, zerosH)
    y2 = _tc2_call(s1, y1, dinv, b1.reshape(1, H), scale, W2)
    s2 = _make_seg_call(O, K, CH)(y2, srcp, dstp, zerosO)
    out = _tc3_call(s2, y2, dinv, b2.reshape(1, O))
    return out[:N]


# speedup vs baseline: 1.1536x; 1.1536x over previous
import jax
import jax.numpy as jnp
from jax import lax
from jax.experimental import pallas as pl
from jax.experimental.pallas import tpu as pltpu
from jax.experimental.pallas import tpu_sc as plsc

N = 10000
E = 320000
F = 128
H = 128
O = 64

NC = 2
NS = 16
NW = NC * NS
K = 128
EW = E // NW
CH = 80
EWP = CH * K
E_PAD = NW * EWP
N_PAD = 10112
RT = N_PAD // NS

import functools


@functools.lru_cache(maxsize=None)
def _mesh():
    return plsc.VectorSubcoreMesh(
        core_axis_name="c", subcore_axis_name="s",
        num_cores=NC, num_subcores=NS)


def _deg_body(dst_hbm, zeros16_hbm, ones_hbm, out_hbm, idx_v, ones_v, acc):
    cid = lax.axis_index("c")
    sid = lax.axis_index("s")
    wid = sid * NC + cid
    pltpu.sync_copy(dst_hbm.at[wid], idx_v)
    pltpu.sync_copy(ones_hbm, ones_v)
    pltpu.sync_copy(zeros16_hbm.at[pl.ds(sid * RT, RT)],
                    acc.at[pl.ds(sid * RT, RT)])
    plsc.subcore_barrier()

    def body(j, carry):
        pltpu.sync_copy(ones_v, acc.at[idx_v.at[j]], add=True)
        return carry

    lax.fori_loop(0, CH, body, 0)
    plsc.subcore_barrier()
    pltpu.sync_copy(acc.at[pl.ds(sid * RT, RT)],
                    out_hbm.at[cid, pl.ds(sid * RT, RT)])


@functools.lru_cache(maxsize=None)
def _deg_call():
    return pl.kernel(
        _deg_body,
        out_type=jax.ShapeDtypeStruct((NC, N_PAD, 16), jnp.float32),
        mesh=_mesh(),
        compiler_params=pltpu.CompilerParams(use_tc_tiling_on_sc=False),
        scratch_types=[
            pltpu.VMEM((CH, K), jnp.int32),
            pltpu.VMEM((K, 16), jnp.float32),
            pltpu.VMEM_SHARED((N_PAD, 16), jnp.float32),
        ],
    )


def _make_seg_call(d, k, ch):
    nbuf = 3

    def body(y_hbm, src_hbm, dst_hbm, zeros_hbm, out_hbm,
             sidx, didx, rows0, rows1, rows2,
             gs0, gs1, gs2, ss0, ss1, ss2, acc):
        rows = (rows0, rows1, rows2)
        gsem = (gs0, gs1, gs2)
        ssem = (ss0, ss1, ss2)
        cid = lax.axis_index("c")
        sid = lax.axis_index("s")
        wid = sid * NC + cid
        pltpu.sync_copy(src_hbm.at[wid], sidx)
        pltpu.sync_copy(dst_hbm.at[wid], didx)
        pltpu.sync_copy(zeros_hbm.at[pl.ds(sid * RT, RT)],
                        acc.at[pl.ds(sid * RT, RT)])
        plsc.subcore_barrier()

        def gather(c, b):
            pltpu.async_copy(y_hbm.at[sidx.at[c]], rows[b], gsem[b])

        def wait_gather(c, b):
            pltpu.make_async_copy(y_hbm.at[sidx.at[c]], rows[b], gsem[b]).wait()

        def scatter(c, b):
            pltpu.async_copy(rows[b], acc.at[didx.at[c]], ssem[b], add=True)

        def wait_scatter(c, b):
            pltpu.make_async_copy(rows[b], acc.at[didx.at[c]], ssem[b]).wait()

        for b in range(nbuf):
            gather(b, b)

        def step(jj, carry):
            for i in range(nbuf):
                c = jj * nbuf + i
                wait_gather(c, i)
                scatter(c, i)
                bp = (i - 1) % nbuf

                @pl.when((c >= 1) & (c + 2 < ch))
                def _():
                    wait_scatter(c - 1, bp)
                    gather(c + 2, bp)

            return carry

        lax.fori_loop(0, ch // nbuf, step, 0)
        for i in range(nbuf):
            c = ch - nbuf + i
            wait_scatter(c, c % nbuf)
        plsc.subcore_barrier()
        pltpu.sync_copy(acc.at[pl.ds(sid * RT, RT)],
                        out_hbm.at[cid, pl.ds(sid * RT, RT)])

    return pl.kernel(
        body,
        out_type=jax.ShapeDtypeStruct((NC, N_PAD, d), jnp.float32),
        mesh=_mesh(),
        compiler_params=pltpu.CompilerParams(use_tc_tiling_on_sc=False),
        scratch_types=[
            pltpu.VMEM((ch, k), jnp.int32),
            pltpu.VMEM((ch, k), jnp.int32),
            pltpu.VMEM((k, d), jnp.float32),
            pltpu.VMEM((k, d), jnp.float32),
            pltpu.VMEM((k, d), jnp.float32),
            pltpu.SemaphoreType.DMA,
            pltpu.SemaphoreType.DMA,
            pltpu.SemaphoreType.DMA,
            pltpu.SemaphoreType.DMA,
            pltpu.SemaphoreType.DMA,
            pltpu.SemaphoreType.DMA,
            pltpu.VMEM_SHARED((N_PAD, d), jnp.float32),
        ],
    )


_make_seg_call = functools.lru_cache(maxsize=None)(_make_seg_call)


def _tc1_body(x_ref, w_ref, degp_ref, y_ref, dinv_ref):
    deg = degp_ref[0][:, 0:1] + degp_ref[1][:, 0:1] + 1.0
    dinv = lax.rsqrt(deg)
    y_ref[...] = dinv * jnp.dot(x_ref[...], w_ref[...],
                                preferred_element_type=jnp.float32)
    dinv_ref[...] = dinv


_tc1_call = pl.pallas_call(
    _tc1_body,
    out_shape=(
        jax.ShapeDtypeStruct((N_PAD, H), jnp.float32),
        jax.ShapeDtypeStruct((N_PAD, 1), jnp.float32),
    ),
)


def _tc2_body(s_ref, y1_ref, dinv_ref, b1_ref, scale_ref, w2_ref, y2_ref):
    dinv = dinv_ref[...]
    h = dinv * (s_ref[0] + s_ref[1] + y1_ref[...]) + b1_ref[...]
    h = jnp.where(h >= 0.0, h, 0.01 * h)
    h = h * scale_ref[...]
    y2_ref[...] = dinv * jnp.dot(h, w2_ref[...],
                                 preferred_element_type=jnp.float32)


_tc2_call = pl.pallas_call(
    _tc2_body,
    out_shape=jax.ShapeDtypeStruct((N_PAD, O), jnp.float32),
)


def _tc3_body(s_ref, y2_ref, dinv_ref, b2_ref, out_ref):
    t = dinv_ref[...] * (s_ref[0] + s_ref[1] + y2_ref[...]) + b2_ref[...]
    out_ref[...] = jnp.where(t >= 0.0, t, 0.01 * t)


_tc3_call = pl.pallas_call(
    _tc3_body,
    out_shape=jax.ShapeDtypeStruct((N_PAD, O), jnp.float32),
)


def kernel(x, edge_idx, W1, b1, W2, b2):
    src = edge_idx[0]
    dst = edge_idx[1]

    def chunked(a, k, ch):
        e_pad = NW * ch * k
        padi = jnp.full((e_pad - E,), N, jnp.int32)
        return jnp.concatenate([a, padi]).reshape(NW, ch, k)

    srcp = chunked(src, K, CH)
    dstp = chunked(dst, K, CH)
    k1, ch1 = 64, 159
    k2, ch2 = 128, 81
    srcp1, dstp1 = chunked(src, k1, ch1), chunked(dst, k1, ch1)
    srcp2, dstp2 = chunked(src, k2, ch2), chunked(dst, k2, ch2)
    xp = jnp.pad(x, ((0, N_PAD - N), (0, 0)))

    zeros16 = jnp.zeros((N_PAD, 16), jnp.float32)
    ones16 = jnp.ones((K, 16), jnp.float32)
    zerosH = jnp.zeros((N_PAD, H), jnp.float32)
    zerosO = jnp.zeros((N_PAD, O), jnp.float32)

    mask = jax.random.bernoulli(jax.random.key(42), 0.5, (N, H))
    scale = jnp.pad(jnp.where(mask, 2.0, 0.0).astype(jnp.float32),
                    ((0, N_PAD - N), (0, 0)))

    degp = _deg_call()(dstp, zeros16, ones16)
    y1, dinv = _tc1_call(xp, W1, degp)
    s1 = _make_seg_call(H, k1, ch1)(y1, srcp1, dstp1, zerosH)
    y2 = _tc2_call(s1, y1, dinv, b1.reshape(1, H), scale, W2)
    s2 = _make_seg_call(O, k2, ch2)(y2, srcp2, dstp2, zerosO)
    out = _tc3_call(s2, y2, dinv, b2.reshape(1, O))
    return out[:N]


# speedup vs baseline: 1.9126x; 1.6579x over previous
import functools

import jax
import jax.numpy as jnp
from jax import lax
from jax.experimental import pallas as pl
from jax.experimental.pallas import tpu as pltpu
from jax.experimental.pallas import tpu_sc as plsc

N = 10000
E = 320000
F = 128
H = 128
O = 64
HD = H // 2

NC = 2
NS = 16
NW = NC * NS
K = 128
CH1 = 157
CH2 = 80
CHD = 80
N_PAD = 10112
RT = N_PAD // NS


@functools.lru_cache(maxsize=None)
def _mesh():
    return plsc.VectorSubcoreMesh(
        core_axis_name="c", subcore_axis_name="s",
        num_cores=NC, num_subcores=NS)


_SC_PARAMS = pltpu.CompilerParams(use_tc_tiling_on_sc=False)


def _deg_body(dst_hbm, zeros16_hbm, ones_hbm, out_hbm, idx_v, ones_v, acc):
    cid = lax.axis_index("c")
    sid = lax.axis_index("s")
    wid = sid * NC + cid
    pltpu.sync_copy(dst_hbm.at[wid], idx_v)
    pltpu.sync_copy(ones_hbm, ones_v)
    pltpu.sync_copy(zeros16_hbm.at[pl.ds(sid * RT, RT)],
                    acc.at[pl.ds(sid * RT, RT)])
    plsc.subcore_barrier()

    def body(j, carry):
        pltpu.sync_copy(ones_v, acc.at[idx_v.at[j]], add=True)
        return carry

    lax.fori_loop(0, CHD, body, 0)
    plsc.subcore_barrier()
    pltpu.sync_copy(acc.at[pl.ds(sid * RT, RT)],
                    out_hbm.at[cid, pl.ds(sid * RT, RT)])


@functools.lru_cache(maxsize=None)
def _deg_call():
    return pl.kernel(
        _deg_body,
        out_type=jax.ShapeDtypeStruct((NC, N_PAD, 16), jnp.float32),
        mesh=_mesh(),
        compiler_params=_SC_PARAMS,
        scratch_types=[
            pltpu.VMEM((CHD, K), jnp.int32),
            pltpu.VMEM((K, 16), jnp.float32),
            pltpu.VMEM_SHARED((N_PAD, 16), jnp.float32),
        ],
    )


@functools.lru_cache(maxsize=None)
def _seg1_call():
    d, k, ch = HD, K, CH1

    def body(y_hbm, src_hbm, dst_hbm, zeros_hbm, out_hbm,
             sidx, didx, rows, ystage, acc):
        cid = lax.axis_index("c")
        sid = lax.axis_index("s")
        pltpu.sync_copy(src_hbm.at[sid], sidx)
        pltpu.sync_copy(dst_hbm.at[sid], didx)
        pltpu.sync_copy(zeros_hbm.at[pl.ds(sid * RT, RT)],
                        acc.at[pl.ds(sid * RT, RT)])
        pltpu.sync_copy(y_hbm.at[cid, pl.ds(sid * RT, RT)],
                        ystage.at[pl.ds(sid * RT, RT)])
        plsc.subcore_barrier()

        def step(j, carry):
            pltpu.sync_copy(ystage.at[sidx.at[j]], rows)
            pltpu.sync_copy(rows, acc.at[didx.at[j]], add=True)
            return carry

        lax.fori_loop(0, ch, step, 0)
        plsc.subcore_barrier()
        pltpu.sync_copy(acc.at[pl.ds(sid * RT, RT)],
                        out_hbm.at[cid, pl.ds(sid * RT, RT)])

    return pl.kernel(
        body,
        out_type=jax.ShapeDtypeStruct((NC, N_PAD, d), jnp.float32),
        mesh=_mesh(),
        compiler_params=_SC_PARAMS,
        scratch_types=[
            pltpu.VMEM((ch, k), jnp.int32),
            pltpu.VMEM((ch, k), jnp.int32),
            pltpu.VMEM((k, d), jnp.float32),
            pltpu.VMEM_SHARED((N_PAD, d), jnp.float32),
            pltpu.VMEM_SHARED((N_PAD, d), jnp.float32),
        ],
    )


@functools.lru_cache(maxsize=None)
def _seg2_call():
    d, k, ch = O, K, CH2

    def body(y_hbm, src_hbm, dst_hbm, zeros_hbm, out_hbm,
             sidx, didx, rows, ystage, acc):
        cid = lax.axis_index("c")
        sid = lax.axis_index("s")
        wid = sid * NC + cid
        pltpu.sync_copy(src_hbm.at[wid], sidx)
        pltpu.sync_copy(dst_hbm.at[wid], didx)
        pltpu.sync_copy(zeros_hbm.at[pl.ds(sid * RT, RT)],
                        acc.at[pl.ds(sid * RT, RT)])
        pltpu.sync_copy(y_hbm.at[pl.ds(sid * RT, RT)],
                        ystage.at[pl.ds(sid * RT, RT)])
        plsc.subcore_barrier()

        def step(j, carry):
            pltpu.sync_copy(ystage.at[sidx.at[j]], rows)
            pltpu.sync_copy(rows, acc.at[didx.at[j]], add=True)
            return carry

        lax.fori_loop(0, ch, step, 0)
        plsc.subcore_barrier()
        pltpu.sync_copy(acc.at[pl.ds(sid * RT, RT)],
                        out_hbm.at[cid, pl.ds(sid * RT, RT)])

    return pl.kernel(
        body,
        out_type=jax.ShapeDtypeStruct((NC, N_PAD, d), jnp.float32),
        mesh=_mesh(),
        compiler_params=_SC_PARAMS,
        scratch_types=[
            pltpu.VMEM((ch, k), jnp.int32),
            pltpu.VMEM((ch, k), jnp.int32),
            pltpu.VMEM((k, d), jnp.float32),
            pltpu.VMEM_SHARED((N_PAD, d), jnp.float32),
            pltpu.VMEM_SHARED((N_PAD, d), jnp.float32),
        ],
    )


def _tc1_body(x_ref, w_ref, degp_ref, y_ref, dinv_ref):
    deg = degp_ref[0][:, 0:1] + degp_ref[1][:, 0:1] + 1.0
    dinv = lax.rsqrt(deg)
    y = dinv * jnp.dot(x_ref[...], w_ref[...],
                       preferred_element_type=jnp.float32)
    y_ref[0] = y[:, :HD]
    y_ref[1] = y[:, HD:]
    dinv_ref[...] = dinv


_tc1_call = pl.pallas_call(
    _tc1_body,
    out_shape=(
        jax.ShapeDtypeStruct((NC, N_PAD, HD), jnp.float32),
        jax.ShapeDtypeStruct((N_PAD, 1), jnp.float32),
    ),
)


def _tc2_body(s_ref, y1_ref, dinv_ref, b1_ref, scale_ref, w2_ref, y2_ref):
    dinv = dinv_ref[...]
    s1 = jnp.concatenate([s_ref[0], s_ref[1]], axis=-1)
    y1 = jnp.concatenate([y1_ref[0], y1_ref[1]], axis=-1)
    h = dinv * (s1 + y1) + b1_ref[...]
    h = jnp.where(h >= 0.0, h, 0.01 * h)
    h = h * scale_ref[...]
    y2_ref[...] = dinv * jnp.dot(h, w2_ref[...],
                                 preferred_element_type=jnp.float32)


_tc2_call = pl.pallas_call(
    _tc2_body,
    out_shape=jax.ShapeDtypeStruct((N_PAD, O), jnp.float32),
)


def _tc3_body(s_ref, y2_ref, dinv_ref, b2_ref, out_ref):
    t = dinv_ref[...] * (s_ref[0] + s_ref[1] + y2_ref[...]) + b2_ref[...]
    out_ref[...] = jnp.where(t >= 0.0, t, 0.01 * t)


_tc3_call = pl.pallas_call(
    _tc3_body,
    out_shape=jax.ShapeDtypeStruct((N_PAD, O), jnp.float32),
)


def kernel(x, edge_idx, W1, b1, W2, b2):
    src = edge_idx[0]
    dst = edge_idx[1]

    def chunked(a, t, ch):
        padi = jnp.full((t * ch * K - E,), N, jnp.int32)
        return jnp.concatenate([a, padi]).reshape(t, ch, K)

    srcp1 = chunked(src, NS, CH1)
    dstp1 = chunked(dst, NS, CH1)
    srcp2 = chunked(src, NW, CH2)
    dstp2 = chunked(dst, NW, CH2)
    dstpd = chunked(dst, NW, CHD)
    xp = jnp.pad(x, ((0, N_PAD - N), (0, 0)))

    zeros16 = jnp.zeros((N_PAD, 16), jnp.float32)
    ones16 = jnp.ones((K, 16), jnp.float32)
    zerosD = jnp.zeros((N_PAD, HD), jnp.float32)

    mask = jax.random.bernoulli(jax.random.key(42), 0.5, (N, H))
    scale = jnp.pad(jnp.where(mask, 2.0, 0.0).astype(jnp.float32),
                    ((0, N_PAD - N), (0, 0)))

    degp = _deg_call()(dstpd, zeros16, ones16)
    y1h, dinv = _tc1_call(xp, W1, degp)
    s1 = _seg1_call()(y1h, srcp1, dstp1, zerosD)
    y2 = _tc2_call(s1, y1h, dinv, b1.reshape(1, H), scale, W2)
    s2 = _seg2_call()(y2, srcp2, dstp2, zerosD)
    out = _tc3_call(s2, y2, dinv, b2.reshape(1, O))
    return out[:N]


# speedup vs baseline: 2.6184x; 1.3690x over previous
import functools

import jax
import jax.numpy as jnp
from jax import lax
from jax.experimental import pallas as pl
from jax.experimental.pallas import tpu as pltpu
from jax.experimental.pallas import tpu_sc as plsc

N = 10000
E = 320000
F = 128
H = 128
O = 64
HD = H // 2

NC = 2
NS = 16
NW = NC * NS
K = 128
CH1 = 162
HF1 = CH1 // 2
CH2 = 81
CHD = 80
N_PAD = 10112
RT = N_PAD // NS


@functools.lru_cache(maxsize=None)
def _mesh():
    return plsc.VectorSubcoreMesh(
        core_axis_name="c", subcore_axis_name="s",
        num_cores=NC, num_subcores=NS)


_SC_PARAMS = pltpu.CompilerParams(use_tc_tiling_on_sc=False)

_NBUF = 3


def _run_pipe(ystage, acc, sidx, didx, rows, gsem, ssem, ch):
    def gather(c, b):
        pltpu.async_copy(ystage.at[sidx.at[c]], rows[b], gsem[b])

    def wait_gather(c, b):
        pltpu.make_async_copy(ystage.at[sidx.at[c]], rows[b], gsem[b]).wait()

    def scatter(c, b):
        pltpu.async_copy(rows[b], acc.at[didx.at[c]], ssem[b], add=True)

    def wait_scatter(c, b):
        pltpu.make_async_copy(rows[b], acc.at[didx.at[c]], ssem[b]).wait()

    for b in range(_NBUF):
        gather(b, b)

    def step(jj, carry):
        for i in range(_NBUF):
            c = jj * _NBUF + i
            wait_gather(c, i)
            scatter(c, i)
            bp = (i - 1) % _NBUF

            @pl.when((c >= 1) & (c + 2 < ch))
            def _():
                wait_scatter(c - 1, bp)
                gather(c + 2, bp)

        return carry

    lax.fori_loop(0, ch // _NBUF, step, 0)
    for i in range(_NBUF):
        wait_scatter(ch - _NBUF + i, (ch - _NBUF + i) % _NBUF)


def _deg_body(dst_hbm, zeros16_hbm, ones_hbm, out_hbm, idx_v, ones_v, acc):
    cid = lax.axis_index("c")
    sid = lax.axis_index("s")
    wid = sid * NC + cid
    pltpu.sync_copy(dst_hbm.at[wid], idx_v)
    pltpu.sync_copy(ones_hbm, ones_v)
    pltpu.sync_copy(zeros16_hbm.at[pl.ds(sid * RT, RT)],
                    acc.at[pl.ds(sid * RT, RT)])
    plsc.subcore_barrier()

    def body(j, carry):
        pltpu.sync_copy(ones_v, acc.at[idx_v.at[j]], add=True)
        return carry

    lax.fori_loop(0, CHD, body, 0)
    plsc.subcore_barrier()
    pltpu.sync_copy(acc.at[pl.ds(sid * RT, RT)],
                    out_hbm.at[cid, pl.ds(sid * RT, RT)])


@functools.lru_cache(maxsize=None)
def _deg_call():
    return pl.kernel(
        _deg_body,
        out_type=jax.ShapeDtypeStruct((NC, N_PAD, 16), jnp.float32),
        mesh=_mesh(),
        compiler_params=_SC_PARAMS,
        scratch_types=[
            pltpu.VMEM((CHD, K), jnp.int32),
            pltpu.VMEM((K, 16), jnp.float32),
            pltpu.VMEM_SHARED((N_PAD, 16), jnp.float32),
        ],
    )


@functools.lru_cache(maxsize=None)
def _seg1_call():
    d, k = HD, K

    def body(y_hbm, src_hbm, dst_hbm, zeros_hbm, out_hbm,
             sidx, didx, rows0, rows1, rows2,
             gs0, gs1, gs2, ss0, ss1, ss2, ystage, acc):
        rows = (rows0, rows1, rows2)
        gsem = (gs0, gs1, gs2)
        ssem = (ss0, ss1, ss2)
        cid = lax.axis_index("c")
        sid = lax.axis_index("s")
        pltpu.sync_copy(zeros_hbm.at[pl.ds(sid * RT, RT)],
                        acc.at[pl.ds(sid * RT, RT)])
        pltpu.sync_copy(y_hbm.at[cid, pl.ds(sid * RT, RT)],
                        ystage.at[pl.ds(sid * RT, RT)])
        plsc.subcore_barrier()

        for half in range(2):
            pltpu.sync_copy(src_hbm.at[sid, pl.ds(half * HF1, HF1)], sidx)
            pltpu.sync_copy(dst_hbm.at[sid, pl.ds(half * HF1, HF1)], didx)
            _run_pipe(ystage, acc, sidx, didx, rows, gsem, ssem, HF1)

        plsc.subcore_barrier()
        pltpu.sync_copy(acc.at[pl.ds(sid * RT, RT)],
                        out_hbm.at[cid, pl.ds(sid * RT, RT)])

    return pl.kernel(
        body,
        out_type=jax.ShapeDtypeStruct((NC, N_PAD, d), jnp.float32),
        mesh=_mesh(),
        compiler_params=_SC_PARAMS,
        scratch_types=[
            pltpu.VMEM((HF1, k), jnp.int32),
            pltpu.VMEM((HF1, k), jnp.int32),
            pltpu.VMEM((k, d), jnp.float32),
            pltpu.VMEM((k, d), jnp.float32),
            pltpu.VMEM((k, d), jnp.float32),
            pltpu.SemaphoreType.DMA,
            pltpu.SemaphoreType.DMA,
            pltpu.SemaphoreType.DMA,
            pltpu.SemaphoreType.DMA,
            pltpu.SemaphoreType.DMA,
            pltpu.SemaphoreType.DMA,
            pltpu.VMEM_SHARED((N_PAD, d), jnp.float32),
            pltpu.VMEM_SHARED((N_PAD, d), jnp.float32),
        ],
    )


@functools.lru_cache(maxsize=None)
def _seg2_call():
    d, k, ch = O, K, CH2

    def body(y_hbm, src_hbm, dst_hbm, zeros_hbm, out_hbm,
             sidx, didx, rows0, rows1, rows2,
             gs0, gs1, gs2, ss0, ss1, ss2, ystage, acc):
        rows = (rows0, rows1, rows2)
        gsem = (gs0, gs1, gs2)
        ssem = (ss0, ss1, ss2)
        cid = lax.axis_index("c")
        sid = lax.axis_index("s")
        wid = sid * NC + cid
        pltpu.sync_copy(src_hbm.at[wid], sidx)
        pltpu.sync_copy(dst_hbm.at[wid], didx)
        pltpu.sync_copy(zeros_hbm.at[pl.ds(sid * RT, RT)],
                        acc.at[pl.ds(sid * RT, RT)])
        pltpu.sync_copy(y_hbm.at[pl.ds(sid * RT, RT)],
                        ystage.at[pl.ds(sid * RT, RT)])
        plsc.subcore_barrier()
        _run_pipe(ystage, acc, sidx, didx, rows, gsem, ssem, ch)
        plsc.subcore_barrier()
        pltpu.sync_copy(acc.at[pl.ds(sid * RT, RT)],
                        out_hbm.at[cid, pl.ds(sid * RT, RT)])

    return pl.kernel(
        body,
        out_type=jax.ShapeDtypeStruct((NC, N_PAD, d), jnp.float32),
        mesh=_mesh(),
        compiler_params=_SC_PARAMS,
        scratch_types=[
            pltpu.VMEM((ch, k), jnp.int32),
            pltpu.VMEM((ch, k), jnp.int32),
            pltpu.VMEM((k, d), jnp.float32),
            pltpu.VMEM((k, d), jnp.float32),
            pltpu.VMEM((k, d), jnp.float32),
            pltpu.SemaphoreType.DMA,
            pltpu.SemaphoreType.DMA,
            pltpu.SemaphoreType.DMA,
            pltpu.SemaphoreType.DMA,
            pltpu.SemaphoreType.DMA,
            pltpu.SemaphoreType.DMA,
            pltpu.VMEM_SHARED((N_PAD, d), jnp.float32),
            pltpu.VMEM_SHARED((N_PAD, d), jnp.float32),
        ],
    )


def _tc1_body(x_ref, w_ref, degp_ref, y_ref, dinv_ref):
    deg = degp_ref[0][:, 0:1] + degp_ref[1][:, 0:1] + 1.0
    dinv = lax.rsqrt(deg)
    y = dinv * jnp.dot(x_ref[...], w_ref[...],
                       preferred_element_type=jnp.float32)
    y_ref[0] = y[:, :HD]
    y_ref[1] = y[:, HD:]
    dinv_ref[...] = dinv


_tc1_call = pl.pallas_call(
    _tc1_body,
    out_shape=(
        jax.ShapeDtypeStruct((NC, N_PAD, HD), jnp.float32),
        jax.ShapeDtypeStruct((N_PAD, 1), jnp.float32),
    ),
)


def _tc2_body(s_ref, y1_ref, dinv_ref, b1_ref, scale_ref, w2_ref, y2_ref):
    dinv = dinv_ref[...]
    s1 = jnp.concatenate([s_ref[0], s_ref[1]], axis=-1)
    y1 = jnp.concatenate([y1_ref[0], y1_ref[1]], axis=-1)
    h = dinv * (s1 + y1) + b1_ref[...]
    h = jnp.where(h >= 0.0, h, 0.01 * h)
    h = h * scale_ref[...]
    y2_ref[...] = dinv * jnp.dot(h, w2_ref[...],
                                 preferred_element_type=jnp.float32)


_tc2_call = pl.pallas_call(
    _tc2_body,
    out_shape=jax.ShapeDtypeStruct((N_PAD, O), jnp.float32),
)


def _tc3_body(s_ref, y2_ref, dinv_ref, b2_ref, out_ref):
    t = dinv_ref[...] * (s_ref[0] + s_ref[1] + y2_ref[...]) + b2_ref[...]
    out_ref[...] = jnp.where(t >= 0.0, t, 0.01 * t)


_tc3_call = pl.pallas_call(
    _tc3_body,
    out_shape=jax.ShapeDtypeStruct((N_PAD, O), jnp.float32),
)


def kernel(x, edge_idx, W1, b1, W2, b2):
    src = edge_idx[0]
    dst = edge_idx[1]

    def chunked(a, t, ch):
        padi = jnp.full((t * ch * K - E,), N, jnp.int32)
        return jnp.concatenate([a, padi]).reshape(t, ch, K)

    srcp1 = chunked(src, NS, CH1)
    dstp1 = chunked(dst, NS, CH1)
    srcp2 = chunked(src, NW, CH2)
    dstp2 = chunked(dst, NW, CH2)
    dstpd = chunked(dst, NW, CHD)
    xp = jnp.pad(x, ((0, N_PAD - N), (0, 0)))

    zeros16 = jnp.zeros((N_PAD, 16), jnp.float32)
    ones16 = jnp.ones((K, 16), jnp.float32)
    zerosD = jnp.zeros((N_PAD, HD), jnp.float32)

    mask = jax.random.bernoulli(jax.random.key(42), 0.5, (N, H))
    scale = jnp.pad(jnp.where(mask, 2.0, 0.0).astype(jnp.float32),
                    ((0, N_PAD - N), (0, 0)))

    degp = _deg_call()(dstpd, zeros16, ones16)
    y1h, dinv = _tc1_call(xp, W1, degp)
    s1 = _seg1_call()(y1h, srcp1, dstp1, zerosD)
    y2 = _tc2_call(s1, y1h, dinv, b1.reshape(1, H), scale, W2)
    s2 = _seg2_call()(y2, srcp2, dstp2, zerosD)
    out = _tc3_call(s2, y2, dinv, b2.reshape(1, O))
    return out[:N]
